# baseline (speedup 1.0000x reference)
import jax
import jax.numpy as jnp
from jax import lax
from jax.experimental import pallas as pl
from jax.experimental.pallas import tpu as pltpu
from jax.experimental.pallas import tpu_sc as plsc

_N = 10000
_E = 320000
_D = 128
_NC = 2
_NS = 16
_TILES = _NC * _NS
_B = 128
_NB = 80
_EP = _TILES * _NB * _B
_SLOTS = 1
_NP = 10240
_RPT = _NP // _NS
_ZB = _B
_VPR = _D // 16


def _spmm_body(h_hbm, rows_hbm, cols_hbm, vals_hbm, out_hbm, *refs):
    cols_s = refs[0:_SLOTS]
    ridx_s = refs[_SLOTS:2 * _SLOTS]
    vals_s = refs[2 * _SLOTS:3 * _SLOTS]
    buf16_s = refs[3 * _SLOTS:4 * _SLOTS]
    msg_s = refs[4 * _SLOTS:5 * _SLOTS]
    acc_sh = refs[5 * _SLOTS]
    sems = refs[5 * _SLOTS + 1:5 * _SLOTS + 1 + _SLOTS]

    c = lax.axis_index("c")
    s = lax.axis_index("s")
    tid = c * _NS + s

    zbuf = msg_s[0]
    def _zrow(i, carry):
        for j in range(_VPR):
            zbuf[i, pl.ds(j * 16, 16)] = jnp.zeros((16,), jnp.float32)
        return carry
    lax.fori_loop(0, _ZB, _zrow, 0)
    for k in range(_RPT // _ZB):
        pltpu.sync_copy(zbuf, acc_sh.at[pl.ds(s * _RPT + k * _ZB, _ZB)])
    plsc.subcore_barrier()

    ebase = tid * _NB * _B

    def _prime(slot, b):
        base = ebase + b * _B
        pltpu.sync_copy(cols_hbm.at[pl.ds(base, _B)], cols_s[slot])
        pltpu.sync_copy(rows_hbm.at[pl.ds(base, _B)], ridx_s[slot])
        pltpu.sync_copy(vals_hbm.at[pl.ds(base, _B)], vals_s[slot])
        pltpu.async_copy(h_hbm.at[cols_s[slot]], buf16_s[slot], sems[slot])

    def _proc(slot):
        b16 = buf16_s[slot]
        buf = msg_s[slot]
        pltpu.make_async_copy(h_hbm.at[cols_s[slot]], b16, sems[slot]).wait()

        def _scale(g, carry):
            vv = vals_s[slot][pl.ds(g * 16, 16)]
            for k in range(16):
                v = vv[k]
                r = g * 16 + k
                for j in range(_D // 32):
                    y = b16[r, pl.ds(j * 16, 16)]
                    lo = plsc.bitcast(y << 16, jnp.float32)
                    hi = plsc.bitcast(y & jnp.int32(-65536), jnp.float32)
                    buf[r, pl.ds(j * 32, 16)] = lo * v
                    buf[r, pl.ds(j * 32 + 16, 16)] = hi * v
            return carry
        lax.fori_loop(0, _B // 16, _scale, 0)
        pltpu.sync_copy(buf, acc_sh.at[ridx_s[slot]], add=True)

    for slot in range(_SLOTS):
        _prime(slot, slot)

    def _round(i, carry):
        b0 = _SLOTS * i
        for slot in range(_SLOTS):
            _proc(slot)
            _prime(slot, b0 + slot + _SLOTS)
        return carry
    lax.fori_loop(0, _NB // _SLOTS - 1, _round, 0)
    for slot in range(_SLOTS):
        _proc(slot)

    plsc.subcore_barrier()
    for k in range(_RPT // _ZB):
        r0 = s * _RPT + k * _ZB
        pltpu.sync_copy(acc_sh.at[pl.ds(r0, _ZB)], zbuf)
        pltpu.sync_copy(zbuf, out_hbm.at[c, pl.ds(r0, _ZB)])


def _spmm(h, rows_p, cols_p, vals_p):
    mesh = plsc.VectorSubcoreMesh(
        core_axis_name="c", subcore_axis_name="s",
        num_cores=_NC, num_subcores=_NS)
    scratch = (
        [pltpu.VMEM((_B,), jnp.int32) for _ in range(_SLOTS)]
        + [pltpu.VMEM((_B,), jnp.int32) for _ in range(_SLOTS)]
        + [pltpu.VMEM((_B,), jnp.float32) for _ in range(_SLOTS)]
        + [pltpu.VMEM((_B, _D // 2), jnp.int32) for _ in range(_SLOTS)]
        + [pltpu.VMEM((_B, _D), jnp.float32) for _ in range(_SLOTS)]
        + [pltpu.VMEM_SHARED((_NP, _D), jnp.float32)]
        + [pltpu.SemaphoreType.DMA for _ in range(_SLOTS)]
    )
    return pl.kernel(
        _spmm_body,
        out_type=jax.ShapeDtypeStruct((_NC, _NP, _D), jnp.float32),
        mesh=mesh,
        scratch_types=scratch,
        compiler_params=pltpu.CompilerParams(
            use_tc_tiling_on_sc=False, needs_layout_passes=False),
    )(h, rows_p, cols_p, vals_p)


_BLK = 1000


def _layer_body(xa, xb, w, b, o):
    x = xa[0] + xb[0]
    y = jnp.dot(x, w[...], preferred_element_type=jnp.float32) + b[...]
    o[...] = jnp.where(y >= 0, y, 0.2 * y).astype(jnp.bfloat16)


def _layer(parts, w, b):
    return pl.pallas_call(
        _layer_body,
        grid=(_N // _BLK,),
        in_specs=[
            pl.BlockSpec((1, _BLK, _D), lambda i: (0, i, 0)),
            pl.BlockSpec((1, _BLK, _D), lambda i: (1, i, 0)),
            pl.BlockSpec((_D, _D), lambda i: (0, 0)),
            pl.BlockSpec((1, _D), lambda i: (0, 0)),
        ],
        out_specs=pl.BlockSpec((_BLK, _D), lambda i: (i, 0)),
        out_shape=jax.ShapeDtypeStruct((_N, _D), jnp.bfloat16),
    )(parts, parts, w, b.reshape(1, _D))


def _final_body(xa, xb, w1, b1, wmu, bmu, wlv, blv, ini,
                tuned_o, mu_o, lv_o):
    x = xa[0] + xb[0]
    h = jnp.dot(x, w1[...], preferred_element_type=jnp.float32) + b1[...]
    h = jnp.where(h >= 0, h, 0.2 * h)
    mu = jnp.dot(h, wmu[...], preferred_element_type=jnp.float32) + bmu[...]
    lv = jnp.dot(h, wlv[...], preferred_element_type=jnp.float32) + blv[...]
    mu_o[...] = mu
    lv_o[...] = jnp.clip(lv, -20.0, 20.0)
    tuned_o[...] = ini[...] + jnp.where(mu >= 0, mu, 0.25 * mu)


def _final(parts, w1, b1, wmu, bmu, wlv, blv, ini):
    full = pl.BlockSpec((_D, _D), lambda i: (0, 0))
    vec = pl.BlockSpec((1, _D), lambda i: (0, 0))
    blk = pl.BlockSpec((_BLK, _D), lambda i: (i, 0))
    return pl.pallas_call(
        _final_body,
        grid=(_N // _BLK,),
        in_specs=[
            pl.BlockSpec((1, _BLK, _D), lambda i: (0, i, 0)),
            pl.BlockSpec((1, _BLK, _D), lambda i: (1, i, 0)),
            full, vec, full, vec, full, vec, blk,
        ],
        out_specs=(blk, blk, blk),
        out_shape=(
            jax.ShapeDtypeStruct((_N, _D), jnp.float32),
            jax.ShapeDtypeStruct((_N, _D), jnp.float32),
            jax.ShapeDtypeStruct((_N, _D), jnp.float32),
        ),
    )(parts, parts, w1, b1.reshape(1, _D), wmu, bmu.reshape(1, _D),
      wlv, blv.reshape(1, _D), ini)


@jax.jit
def kernel(edge_index, edge_vals, node_feats, ini_embeds,
           W0, b0, W1, b1, Wmu, bmu, Wlv, blv):
    pad = _EP - _E
    rows_p = jnp.concatenate([edge_index[0], jnp.full((pad,), _N, jnp.int32)])
    cols_p = jnp.concatenate([edge_index[1], jnp.zeros((pad,), jnp.int32)])
    vals_p = jnp.concatenate([edge_vals, jnp.zeros((pad,), jnp.float32)])

    perm = jnp.arange(_D).reshape(_D // 32, 2, 16).transpose(0, 2, 1).reshape(_D)
    nf16 = node_feats[:, perm].astype(jnp.bfloat16)
    nf32 = lax.bitcast_convert_type(
        nf16.reshape(_N, _D // 2, 2), jnp.int32)
    W0c = W0[:, perm]
    b0c = b0[perm]

    s1 = _spmm(nf32, rows_p, cols_p, vals_p)
    h1 = _layer(s1, W0c, b0c)
    h1_32 = lax.bitcast_convert_type(
        h1.reshape(_N, _D // 2, 2), jnp.int32)
    s2 = _spmm(h1_32, rows_p, cols_p, vals_p)
    return _final(s2, W1, b1, Wmu, bmu, Wlv, blv, ini_embeds)


# speedup vs baseline: 1.4578x; 1.4578x over previous
import jax
import jax.numpy as jnp
from jax import lax
from jax.experimental import pallas as pl
from jax.experimental.pallas import tpu as pltpu
from jax.experimental.pallas import tpu_sc as plsc

_N = 10000
_E = 320000
_D = 128
_NC = 2
_NS = 16
_TILES = _NC * _NS
_B = 128
_NB = 80
_EP = _TILES * _NB * _B
_SLOTS = 2
_NP = 10240
_RPT = _NP // _NS
_ZB = _B
_VPR = _D // 16


def _spmm_body(h_hbm, rows_hbm, cols_hbm, vals_hbm, out_hbm, *refs):
    cols_s = refs[0:_SLOTS]
    ridx_s = refs[_SLOTS:2 * _SLOTS]
    vals_s = refs[2 * _SLOTS:3 * _SLOTS]
    msg_s = refs[3 * _SLOTS:4 * _SLOTS]
    acc_sh = refs[4 * _SLOTS]
    sems = refs[4 * _SLOTS + 1:4 * _SLOTS + 1 + _SLOTS]

    c = lax.axis_index("c")
    s = lax.axis_index("s")
    tid = c * _NS + s

    zbuf = msg_s[0]
    def _zrow(i, carry):
        for j in range(_VPR):
            zbuf[i, pl.ds(j * 16, 16)] = jnp.zeros((16,), jnp.float32)
        return carry
    lax.fori_loop(0, _ZB, _zrow, 0)
    for k in range(_RPT // _ZB):
        pltpu.sync_copy(zbuf, acc_sh.at[pl.ds(s * _RPT + k * _ZB, _ZB)])
    plsc.subcore_barrier()

    ebase = tid * _NB * _B

    def _prime(slot, b):
        base = ebase + b * _B
        pltpu.sync_copy(cols_hbm.at[pl.ds(base, _B)], cols_s[slot])
        pltpu.sync_copy(rows_hbm.at[pl.ds(base, _B)], ridx_s[slot])
        pltpu.sync_copy(vals_hbm.at[pl.ds(base, _B)], vals_s[slot])
        pltpu.async_copy(h_hbm.at[cols_s[slot]], msg_s[slot], sems[slot])

    def _proc(slot):
        buf = msg_s[slot]
        pltpu.make_async_copy(h_hbm.at[cols_s[slot]], buf, sems[slot]).wait()

        def _scale(g, carry):
            vv = vals_s[slot][pl.ds(g * 16, 16)]
            for k in range(16):
                v = vv[k]
                r = g * 16 + k
                for j in range(_VPR):
                    sl = pl.ds(j * 16, 16)
                    buf[r, sl] = buf[r, sl] * v
            return carry
        lax.fori_loop(0, _B // 16, _scale, 0)
        pltpu.sync_copy(buf, acc_sh.at[ridx_s[slot]], add=True)

    for slot in range(_SLOTS):
        _prime(slot, slot)

    def _round(i, carry):
        b0 = _SLOTS * i
        for slot in range(_SLOTS):
            _proc(slot)
            _prime(slot, b0 + slot + _SLOTS)
        return carry
    lax.fori_loop(0, _NB // _SLOTS - 1, _round, 0)
    for slot in range(_SLOTS):
        _proc(slot)

    plsc.subcore_barrier()
    for k in range(_RPT // _ZB):
        r0 = s * _RPT + k * _ZB
        pltpu.sync_copy(acc_sh.at[pl.ds(r0, _ZB)], zbuf)
        pltpu.sync_copy(zbuf, out_hbm.at[c, pl.ds(r0, _ZB)])


def _spmm(h, rows_p, cols_p, vals_p):
    mesh = plsc.VectorSubcoreMesh(
        core_axis_name="c", subcore_axis_name="s",
        num_cores=_NC, num_subcores=_NS)
    scratch = (
        [pltpu.VMEM((_B,), jnp.int32) for _ in range(_SLOTS)]
        + [pltpu.VMEM((_B,), jnp.int32) for _ in range(_SLOTS)]
        + [pltpu.VMEM((_B,), jnp.float32) for _ in range(_SLOTS)]
        + [pltpu.VMEM((_B, _D), jnp.float32) for _ in range(_SLOTS)]
        + [pltpu.VMEM_SHARED((_NP, _D), jnp.float32)]
        + [pltpu.SemaphoreType.DMA for _ in range(_SLOTS)]
    )
    return pl.kernel(
        _spmm_body,
        out_type=jax.ShapeDtypeStruct((_NC, _NP, _D), jnp.float32),
        mesh=mesh,
        scratch_types=scratch,
    )(h, rows_p, cols_p, vals_p)


_BLK = 1000


def _layer_body(xa, xb, w, b, o):
    x = xa[0] + xb[0]
    y = jnp.dot(x, w[...], preferred_element_type=jnp.float32) + b[...]
    o[...] = jnp.where(y >= 0, y, 0.2 * y)


def _layer(parts, w, b):
    return pl.pallas_call(
        _layer_body,
        grid=(_N // _BLK,),
        in_specs=[
            pl.BlockSpec((1, _BLK, _D), lambda i: (0, i, 0)),
            pl.BlockSpec((1, _BLK, _D), lambda i: (1, i, 0)),
            pl.BlockSpec((_D, _D), lambda i: (0, 0)),
            pl.BlockSpec((1, _D), lambda i: (0, 0)),
        ],
        out_specs=pl.BlockSpec((_BLK, _D), lambda i: (i, 0)),
        out_shape=jax.ShapeDtypeStruct((_N, _D), jnp.float32),
    )(parts, parts, w, b.reshape(1, _D))


def _final_body(xa, xb, w1, b1, wmu, bmu, wlv, blv, ini,
                tuned_o, mu_o, lv_o):
    x = xa[0] + xb[0]
    h = jnp.dot(x, w1[...], preferred_element_type=jnp.float32) + b1[...]
    h = jnp.where(h >= 0, h, 0.2 * h)
    mu = jnp.dot(h, wmu[...], preferred_element_type=jnp.float32) + bmu[...]
    lv = jnp.dot(h, wlv[...], preferred_element_type=jnp.float32) + blv[...]
    mu_o[...] = mu
    lv_o[...] = jnp.clip(lv, -20.0, 20.0)
    tuned_o[...] = ini[...] + jnp.where(mu >= 0, mu, 0.25 * mu)


def _final(parts, w1, b1, wmu, bmu, wlv, blv, ini):
    full = pl.BlockSpec((_D, _D), lambda i: (0, 0))
    vec = pl.BlockSpec((1, _D), lambda i: (0, 0))
    blk = pl.BlockSpec((_BLK, _D), lambda i: (i, 0))
    return pl.pallas_call(
        _final_body,
        grid=(_N // _BLK,),
        in_specs=[
            pl.BlockSpec((1, _BLK, _D), lambda i: (0, i, 0)),
            pl.BlockSpec((1, _BLK, _D), lambda i: (1, i, 0)),
            full, vec, full, vec, full, vec, blk,
        ],
        out_specs=(blk, blk, blk),
        out_shape=(
            jax.ShapeDtypeStruct((_N, _D), jnp.float32),
            jax.ShapeDtypeStruct((_N, _D), jnp.float32),
            jax.ShapeDtypeStruct((_N, _D), jnp.float32),
        ),
    )(parts, parts, w1, b1.reshape(1, _D), wmu, bmu.reshape(1, _D),
      wlv, blv.reshape(1, _D), ini)


@jax.jit
def kernel(edge_index, edge_vals, node_feats, ini_embeds,
           W0, b0, W1, b1, Wmu, bmu, Wlv, blv):
    pad = _EP - _E
    rows_p = jnp.concatenate([edge_index[0], jnp.full((pad,), _N, jnp.int32)])
    cols_p = jnp.concatenate([edge_index[1], jnp.zeros((pad,), jnp.int32)])
    vals_p = jnp.concatenate([edge_vals, jnp.zeros((pad,), jnp.float32)])

    s1 = _spmm(node_feats, rows_p, cols_p, vals_p)
    h1 = _layer(s1, W0, b0)
    s2 = _spmm(h1, rows_p, cols_p, vals_p)
    return _final(s2, W1, b1, Wmu, bmu, Wlv, blv, ini_embeds)
